# baseline (speedup 1.0000x reference)
import functools

import jax
import jax.numpy as jnp
import numpy as np
from jax import lax
from jax.experimental import pallas as pl
from jax.experimental.pallas import tpu as pltpu
from jax.experimental.pallas import tpu_sc as plsc

_K = 16
_NUM_GRAPHS = 16
_H = 64
_NEG = float('-inf')
_IMIN = int(np.iinfo(np.int32).min)


def _knn_kernel(x_ref, xt_ref, x2_ref, batch_ref, idx_ref, ok_ref, *, n_pad, rows):
    i = pl.program_id(0)
    xb = x_ref[...]
    mm = jax.lax.dot_general(
        xb, xt_ref[...], (((1,), (0,)), ((), ())),
        preferred_element_type=jnp.float32)
    x2_row = x2_ref[...]
    x2b = x2_ref[0, pl.ds(i * rows, rows)]
    d = (x2b[:, None] - 2.0 * mm) + x2_row
    cols = jax.lax.broadcasted_iota(jnp.int32, (rows, n_pad), 1)
    row_ids = i * rows + jax.lax.broadcasted_iota(jnp.int32, (rows, n_pad), 0)
    bb = batch_ref[0, pl.ds(i * rows, rows)]
    valid = (bb[:, None] == batch_ref[...]) & (row_ids != cols)
    d = jnp.where(valid, d, jnp.inf)
    nv = jnp.sum(valid.astype(jnp.int32), axis=1)
    for t in range(_K):
        a = jnp.argmin(d, axis=1).astype(jnp.int32)
        idx_ref[:, t] = a
        ok_ref[:, t] = (t < nv).astype(jnp.int32)
        if t + 1 < _K:
            d = jnp.where(cols == a[:, None], jnp.inf, d)


def _knn_win_kernel(lohi_ref, x_ref, xt_ref, x2_ref, batch_ref,
                    idx_ref, ok_ref, cv_ref, ci_ref, nv_ref,
                    *, n_pad, rows, tile):
    i = pl.program_id(0)
    ntile = n_pad // tile
    xb = x_ref[...]
    lo = lohi_ref[0, i]
    hi = lohi_ref[1, i]
    cv_ref[...] = jnp.full((rows, _K), jnp.inf, jnp.float32)
    ci_ref[...] = jnp.zeros((rows, _K), jnp.int32)
    nv_ref[...] = jnp.zeros((rows, _K), jnp.int32)
    x2b = x2_ref[0, pl.ds(i * rows, rows)]
    bb = batch_ref[0, pl.ds(i * rows, rows)]
    rid = i * rows + jax.lax.broadcasted_iota(jnp.int32, (rows, tile), 0)
    imax = jnp.int32(np.iinfo(np.int32).max)
    for j in range(ntile):
        @pl.when((lo <= j) & (j <= hi))
        def _(j=j):
            mm = jax.lax.dot_general(
                xb, xt_ref[:, j * tile:(j + 1) * tile],
                (((1,), (0,)), ((), ())), preferred_element_type=jnp.float32)
            x2t = x2_ref[0, j * tile:(j + 1) * tile]
            bt = batch_ref[0, j * tile:(j + 1) * tile]
            d = (x2b[:, None] - 2.0 * mm) + x2t[None, :]
            cols = j * tile + jax.lax.broadcasted_iota(
                jnp.int32, (rows, tile), 1)
            valid = (bb[:, None] == bt[None, :]) & (rid != cols)
            d = jnp.where(valid, d, jnp.inf)
            nv_ref[...] = nv_ref[...] + jnp.sum(
                valid.astype(jnp.int32), axis=1)[:, None]
            cand_v = jnp.concatenate([cv_ref[...], d], axis=1)
            cand_i = jnp.concatenate([ci_ref[...], cols], axis=1)
            for t in range(_K):
                m = jnp.min(cand_v, axis=1)
                eq = cand_v == m[:, None]
                sel = jnp.min(jnp.where(eq, cand_i, imax), axis=1)
                cv_ref[:, t] = m
                ci_ref[:, t] = sel
                cand_v = jnp.where(eq & (cand_i == sel[:, None]),
                                   jnp.inf, cand_v)
    idx_ref[...] = ci_ref[...]
    tv = jax.lax.broadcasted_iota(jnp.int32, (rows, _K), 1)
    ok_ref[...] = (tv < nv_ref[...]).astype(jnp.int32)


def _knn_pallas_win(xp, bp, n_pad, lohi):
    d_feat = xp.shape[1]
    rows = 256
    tile = 1024
    x2p = jnp.sum(xp * xp, axis=1)
    grid = n_pad // rows
    idx, ok = pl.pallas_call(
        functools.partial(_knn_win_kernel, n_pad=n_pad, rows=rows, tile=tile),
        grid=(grid,),
        in_specs=[
            pl.BlockSpec(memory_space=pltpu.SMEM),
            pl.BlockSpec((rows, d_feat), lambda i: (i, 0)),
            pl.BlockSpec((d_feat, n_pad), lambda i: (0, 0)),
            pl.BlockSpec((1, n_pad), lambda i: (0, 0)),
            pl.BlockSpec((1, n_pad), lambda i: (0, 0)),
        ],
        out_specs=[
            pl.BlockSpec((rows, _K), lambda i: (i, 0)),
            pl.BlockSpec((rows, _K), lambda i: (i, 0)),
        ],
        out_shape=[
            jax.ShapeDtypeStruct((n_pad, _K), jnp.int32),
            jax.ShapeDtypeStruct((n_pad, _K), jnp.int32),
        ],
        scratch_shapes=[
            pltpu.VMEM((rows, _K), jnp.float32),
            pltpu.VMEM((rows, _K), jnp.int32),
            pltpu.VMEM((rows, _K), jnp.int32),
        ],
    )(lohi, xp, xp.T, x2p[None, :], bp[None, :])
    return idx, ok


def _block_windows(bp, n, n_pad, rows=256, tile=1024):
    nblk = n_pad // rows
    first = jnp.arange(nblk) * rows
    breal = bp[:n]
    b_first = bp[first]
    last_real = jnp.minimum(first + rows - 1, n - 1)
    b_last = bp[last_real]
    lo_col = jnp.searchsorted(breal, b_first, side='left')
    hi_col = jnp.searchsorted(breal, b_last, side='right') - 1
    real = first < n
    lo_t = jnp.where(real, lo_col // tile, 1).astype(jnp.int32)
    hi_t = jnp.where(real, hi_col // tile, 0).astype(jnp.int32)
    return jnp.stack([lo_t, hi_t])


def _knn_pallas(xp, bp, n_pad):
    d_feat = xp.shape[1]
    rows = 256
    x2p = jnp.sum(xp * xp, axis=1)
    grid = n_pad // rows
    idx, ok = pl.pallas_call(
        functools.partial(_knn_kernel, n_pad=n_pad, rows=rows),
        grid=(grid,),
        in_specs=[
            pl.BlockSpec((rows, d_feat), lambda i: (i, 0)),
            pl.BlockSpec((d_feat, n_pad), lambda i: (0, 0)),
            pl.BlockSpec((1, n_pad), lambda i: (0, 0)),
            pl.BlockSpec((1, n_pad), lambda i: (0, 0)),
        ],
        out_specs=[
            pl.BlockSpec((rows, _K), lambda i: (i, 0)),
            pl.BlockSpec((rows, _K), lambda i: (i, 0)),
        ],
        out_shape=[
            jax.ShapeDtypeStruct((n_pad, _K), jnp.int32),
            jax.ShapeDtypeStruct((n_pad, _K), jnp.int32),
        ],
    )(xp, xp.T, x2p[None, :], bp[None, :])
    return idx, ok


_CHUNK = 128
_NC = 2
_NS = 16
_NW = _NC * _NS


def _sc_gather_rows(table, idx):
    nw, nc = _NW, _NC
    b = idx.shape[0]
    d = table.shape[1]
    per_w = b // nw
    nchunk = per_w // _CHUNK
    assert nchunk * _CHUNK * nw == b and nchunk % 2 == 0
    idx3 = idx.reshape(nw, nchunk, _CHUNK)
    mesh = plsc.VectorSubcoreMesh(core_axis_name="c", subcore_axis_name="s")

    @functools.partial(
        pl.kernel, mesh=mesh,
        compiler_params=pltpu.CompilerParams(use_tc_tiling_on_sc=False),
        out_type=jax.ShapeDtypeStruct((b, d), table.dtype),
        scratch_types=[
            pltpu.VMEM((nchunk, _CHUNK), jnp.int32),
            pltpu.VMEM((2, _CHUNK, d), table.dtype),
            pltpu.SemaphoreType.DMA,
            pltpu.SemaphoreType.DMA,
            pltpu.SemaphoreType.DMA,
        ],
    )
    def k(table_hbm, idx_hbm, out_hbm, idx_v, rows_v, gsem, wsem0, wsem1):
        wid = lax.axis_index("s") * nc + lax.axis_index("c")
        base = wid * per_w
        pltpu.sync_copy(idx_hbm.at[wid], idx_v)
        wsems = (wsem0, wsem1)

        def gat(j, slot):
            return pltpu.make_async_copy(
                table_hbm.at[idx_v.at[j]], rows_v.at[slot], gsem)

        def wr(j, slot):
            return pltpu.make_async_copy(
                rows_v.at[slot],
                out_hbm.at[pl.ds(base + j * _CHUNK, _CHUNK)], wsems[slot])

        gat(0, 0).start()
        npair = nchunk // 2

        def body(i2, carry):
            j0 = i2 * 2
            gat(j0, 0).wait()
            wr(j0, 0).start()

            @pl.when(j0 >= 1)
            def _():
                wr(j0 - 1, 1).wait()

            gat(j0 + 1, 1).start()
            gat(j0 + 1, 1).wait()
            wr(j0 + 1, 1).start()

            @pl.when(j0 + 2 < nchunk)
            def _():
                wr(j0, 0).wait()
                gat(j0 + 2, 0).start()
            return carry

        lax.fori_loop(0, npair, body, 0)
        wr(nchunk - 2, 0).wait()
        wr(nchunk - 1, 1).wait()

    return k(table, idx3)


def _sc_scatter_add_rows(vals, idx, n_rows):
    nc, ns, nw = _NC, _NS, _NW
    b, d = vals.shape
    per_w = b // nw
    nchunk = per_w // _CHUNK
    idx3 = idx.reshape(nw, nchunk, _CHUNK)
    rows_per_tile = n_rows // ns
    mesh = plsc.VectorSubcoreMesh(core_axis_name="c", subcore_axis_name="s")

    @functools.partial(
        pl.kernel, mesh=mesh,
        compiler_params=pltpu.CompilerParams(use_tc_tiling_on_sc=False),
        out_type=jax.ShapeDtypeStruct((nc, n_rows, d), jnp.float32),
        scratch_types=[
            pltpu.VMEM((nchunk, _CHUNK), jnp.int32),
            pltpu.VMEM((2, _CHUNK, d), jnp.float32),
            pltpu.VMEM_SHARED((n_rows, d), jnp.float32),
            pltpu.SemaphoreType.DMA,
        ],
    )
    def k(vals_hbm, idx_hbm, zeros_hbm, out_hbm, idx_v, rows_v, acc_sh, sem):
        cid = lax.axis_index("c")
        sid = lax.axis_index("s")
        wid = sid * nc + cid
        base = wid * per_w
        pltpu.sync_copy(zeros_hbm.at[pl.ds(sid * rows_per_tile, rows_per_tile)],
                        acc_sh.at[pl.ds(sid * rows_per_tile, rows_per_tile)])
        pltpu.sync_copy(idx_hbm.at[wid], idx_v)
        plsc.subcore_barrier()

        def rd(j, slot):
            return pltpu.make_async_copy(
                vals_hbm.at[pl.ds(base + j * _CHUNK, _CHUNK)],
                rows_v.at[slot], sem)

        rd(0, 0).start()

        def body(i2, carry):
            j0 = i2 * 2
            rd(j0, 0).wait()
            rd(j0 + 1, 1).start()
            pltpu.sync_copy(rows_v.at[0], acc_sh.at[idx_v.at[j0]], add=True)
            rd(j0 + 1, 1).wait()

            @pl.when(j0 + 2 < nchunk)
            def _():
                rd(j0 + 2, 0).start()

            pltpu.sync_copy(rows_v.at[1], acc_sh.at[idx_v.at[j0 + 1]], add=True)
            return carry

        lax.fori_loop(0, nchunk // 2, body, 0)
        plsc.subcore_barrier()
        pltpu.sync_copy(
            acc_sh.at[pl.ds(sid * rows_per_tile, rows_per_tile)],
            out_hbm.at[cid].at[pl.ds(sid * rows_per_tile, rows_per_tile)])

    zeros = jnp.zeros((n_rows, d), jnp.float32)
    out = k(vals, idx3, zeros)
    return out[0] + out[1]


def _edge_mlp_kernel(x_ref, xg_ref, ok_ref, nm_ref,
                     w1_ref, b1_ref, w2_ref, b2_ref,
                     outa_ref, hb_ref, *, vb):
    xv = x_ref[...]
    W1 = w1_ref[...]
    B1 = b1_ref[0]
    W2 = w2_ref[...]
    B2 = b2_ref[0]

    def elu(v):
        vn = jnp.minimum(v, 0.0)
        u = jnp.exp(vn)
        um1 = u - 1.0
        lg = jnp.log(jnp.where(u == 1.0, jnp.e, u))
        em1 = jnp.where(u == 1.0, vn, um1 * (vn / jnp.where(lg == 0.0, 1.0, lg)))
        return jnp.where(v > 0, v, em1)

    def mlp2(f):
        h = jnp.dot(f, W1, preferred_element_type=jnp.float32) + B1
        h = elu(h)
        h = jnp.dot(h, W2, preferred_element_type=jnp.float32) + B2
        return elu(h)

    xg3 = xg_ref[...]
    xv3 = jnp.broadcast_to(xv[None], (_K, vb, _H))
    dA3 = xg3 - xv3
    fA = jnp.concatenate([xv3, dA3], axis=2).reshape(_K * vb, 2 * _H)
    fB = jnp.concatenate([xg3, -dA3], axis=2).reshape(_K * vb, 2 * _H)
    h = mlp2(jnp.concatenate([fA, fB], axis=0))
    hA = h[:_K * vb].reshape(_K, vb, _H)
    hB = h[_K * vb:].reshape(_K, vb, _H)
    okm = ok_ref[...]
    nmm = nm_ref[...]
    outa_ref[...] = jnp.sum(hA * okm[:, :, None], axis=0)
    hb_ref[...] = hB * nmm[:, :, None]


def _edge_mlp(xf, xg, okf, nmf, w1, b1, w2, b2, n_pad, vb=256):
    grid = n_pad // vb
    mw = w1.shape[1]
    outa, hb = pl.pallas_call(
        functools.partial(_edge_mlp_kernel, vb=vb),
        grid=(grid,),
        in_specs=[
            pl.BlockSpec((vb, _H), lambda i: (i, 0)),
            pl.BlockSpec((_K, vb, _H), lambda i: (0, i, 0)),
            pl.BlockSpec((_K, vb), lambda i: (0, i)),
            pl.BlockSpec((_K, vb), lambda i: (0, i)),
            pl.BlockSpec((2 * _H, mw), lambda i: (0, 0)),
            pl.BlockSpec((1, mw), lambda i: (0, 0)),
            pl.BlockSpec((mw, _H), lambda i: (0, 0)),
            pl.BlockSpec((1, _H), lambda i: (0, 0)),
        ],
        out_specs=[
            pl.BlockSpec((vb, _H), lambda i: (i, 0)),
            pl.BlockSpec((_K, vb, _H), lambda i: (0, i, 0)),
        ],
        out_shape=[
            jax.ShapeDtypeStruct((n_pad, _H), jnp.float32),
            jax.ShapeDtypeStruct((_K, n_pad, _H), jnp.float32),
        ],
    )(xf, xg, okf, nmf, w1, b1[None, :], w2, b2[None, :])
    return outa, hb


def _graph_round(xf, bt, w1, b1, w2, b2, n_pad, sorted_windows=None):
    if sorted_windows is not None:
        U, OK = _knn_pallas_win(xf, bt, n_pad, sorted_windows)
    else:
        U, OK = _knn_pallas(xf, bt, n_pad)
    UT = U.T
    OKT = OK.T > 0
    uflat = UT.reshape(-1)
    T = jnp.where(OK > 0, U, -1)
    Tg = _sc_gather_rows(T, uflat)
    vids = jnp.tile(jnp.arange(n_pad, dtype=jnp.int32), _K)
    MUT = jnp.any(Tg == vids[:, None], axis=1).reshape(_K, n_pad)
    NM = OKT & ~MUT
    okf = OKT.astype(jnp.float32)
    nmf = NM.astype(jnp.float32)

    xg = _sc_gather_rows(xf, uflat).reshape(_K, n_pad, _H)
    outa, hb = _edge_mlp(xf, xg, okf, nmf, w1, b1, w2, b2, n_pad)
    accb = _sc_scatter_add_rows(hb.reshape(_K * n_pad, _H), uflat, n_pad)
    hout = outa + accb

    deg = jnp.sum(okf, axis=0) + jax.ops.segment_sum(
        nmf.reshape(-1), uflat, num_segments=n_pad)
    invd = 1.0 / jnp.maximum(deg, 1.0)
    htab = jnp.concatenate(
        [hout, jnp.broadcast_to(invd[:, None], (n_pad, 16))], axis=1)
    hgt = _sc_gather_rows(htab, uflat).reshape(_K, n_pad, _H + 16)
    hg = hgt[:, :, :_H]
    invd_g = hgt[:, :, _H]
    ea = jnp.linalg.norm(hg - hout[None], axis=-1)
    w = ea * (invd[None, :] + invd_g)

    wA = jnp.where(OKT, w, _NEG)
    bestA = jnp.max(wA, axis=0)
    wB = jnp.where(NM, w, _NEG).reshape(-1)
    bestB = jax.ops.segment_max(wB, uflat, num_segments=n_pad)
    best = jnp.maximum(bestA, bestB)
    candA = jnp.max(jnp.where(OKT & (w == best[None, :]), UT, -1), axis=0)
    btab = jnp.broadcast_to(best[:, None], (n_pad, 16))
    bg = _sc_gather_rows(btab, uflat)[:, 0]
    candB = jax.ops.segment_max(
        jnp.where(NM.reshape(-1) & (w.reshape(-1) == bg), vids, -1),
        uflat, num_segments=n_pad)
    prop = jnp.maximum(candA, candB)
    prop = jnp.where(prop < 0, -1, prop)
    idxs = jnp.arange(n_pad, dtype=jnp.int32)
    pp = jnp.where(prop >= 0, prop, idxs)
    mutual = (prop >= 0) & (jnp.take(prop, pp) == idxs)
    return hout, prop, mutual


def _max_pool_pair(prop, mutual, xf, bt, act, n_pad):
    idxs = jnp.arange(n_pad, dtype=jnp.int32)
    am = act & mutual
    partner = jnp.where(am, prop, idxs)
    loser = am & (partner < idxs)
    ptab = jnp.concatenate([
        xf,
        lax.bitcast_convert_type(bt, jnp.float32)[:, None],
        act.astype(jnp.float32)[:, None],
        jnp.zeros((n_pad, 14), jnp.float32),
    ], axis=1)
    npad_idx = ((n_pad + 8191) // 8192) * 8192
    pidx = jnp.zeros((npad_idx,), jnp.int32).at[:n_pad].set(partner)
    rows = _sc_gather_rows(ptab, pidx)[:n_pad]
    xpart = rows[:, :_H]
    bpart = lax.bitcast_convert_type(rows[:, _H], jnp.int32)
    apart = rows[:, _H + 1] > 0.5
    xp = jnp.where(loser[:, None], _NEG, jnp.maximum(xf, xpart))
    bp = jnp.where(loser, _IMIN, jnp.maximum(bt, bpart))
    actp = jnp.where(loser, False, act | apart)
    return xp, bp, actp


def kernel(x, batch, datanorm,
           in_w1, in_b1, in_w2, in_b2, in_w3, in_b3,
           c1_w1, c1_b1, c1_w2, c1_b2,
           c2_w1, c2_b1, c2_w2, c2_b2,
           o_w1, o_b1, o_w2, o_b2, o_w3, o_b3):
    n = x.shape[0]
    n_pad = ((n + 1023) // 1024) * 1024
    npd = n_pad - n

    h = datanorm * x
    for W, b in [(in_w1, in_b1), (in_w2, in_b2), (in_w3, in_b3)]:
        h = jax.nn.elu(h @ W + b)

    hp = jnp.concatenate([h, jnp.zeros((npd, _H), jnp.float32)], axis=0)
    bp0 = jnp.concatenate(
        [batch.astype(jnp.int32), -1 - jnp.arange(npd, dtype=jnp.int32)])
    act = jnp.ones((n_pad,), bool)

    win1 = _block_windows(bp0, n, n_pad)
    h1, prop1, mut1 = _graph_round(hp, bp0, c1_w1, c1_b1, c1_w2, c1_b2, n_pad,
                                   sorted_windows=win1)
    h1p, b1p, act1 = _max_pool_pair(prop1, mut1, h1, bp0, act, n_pad)

    hk = jnp.where(act1[:, None], h1p, 0.0)
    bk = jnp.where(act1, b1p,
                   _NUM_GRAPHS + jnp.arange(n_pad, dtype=jnp.int32))
    bk = jnp.where(jnp.arange(n_pad) < n, bk,
                   -1 - jnp.arange(n_pad, dtype=jnp.int32))

    h2, prop2, mut2 = _graph_round(hk, bk, c2_w1, c2_b1, c2_w2, c2_b2, n_pad)
    h2p, b2p, act2 = _max_pool_pair(prop2, mut2, h2, bk, act1, n_pad)

    hs, bs, acts = h2p[:n], b2p[:n], act2[:n]
    hf = jnp.where(acts[:, None], hs, _NEG)
    bf = jnp.where(acts, bs, 0)
    g = jax.ops.segment_max(hf, bf, num_segments=_NUM_GRAPHS)
    g = jnp.where(jnp.isfinite(g), g, 0.0)
    z = jax.nn.elu(g @ o_w1 + o_b1)
    z = jax.nn.elu(z @ o_w2 + o_b2)
    return z @ o_w3 + o_b3


# speedup vs baseline: 1.0229x; 1.0229x over previous
import functools

import jax
import jax.numpy as jnp
import numpy as np
from jax import lax
from jax.experimental import pallas as pl
from jax.experimental.pallas import tpu as pltpu
from jax.experimental.pallas import tpu_sc as plsc

_K = 16
_NUM_GRAPHS = 16
_H = 64
_NEG = float('-inf')
_IMIN = int(np.iinfo(np.int32).min)


def _knn_kernel(x_ref, xt_ref, x2_ref, batch_ref, idx_ref, ok_ref, *, n_pad, rows):
    i = pl.program_id(0)
    xb = x_ref[...]
    mm = jax.lax.dot_general(
        xb, xt_ref[...], (((1,), (0,)), ((), ())),
        preferred_element_type=jnp.float32)
    x2_row = x2_ref[...]
    x2b = x2_ref[0, pl.ds(i * rows, rows)]
    d = (x2b[:, None] - 2.0 * mm) + x2_row
    cols = jax.lax.broadcasted_iota(jnp.int32, (rows, n_pad), 1)
    row_ids = i * rows + jax.lax.broadcasted_iota(jnp.int32, (rows, n_pad), 0)
    bb = batch_ref[0, pl.ds(i * rows, rows)]
    valid = (bb[:, None] == batch_ref[...]) & (row_ids != cols)
    d = jnp.where(valid, d, jnp.inf)
    nv = jnp.sum(valid.astype(jnp.int32), axis=1)
    for t in range(_K):
        a = jnp.argmin(d, axis=1).astype(jnp.int32)
        idx_ref[:, t] = a
        ok_ref[:, t] = (t < nv).astype(jnp.int32)
        if t + 1 < _K:
            d = jnp.where(cols == a[:, None], jnp.inf, d)


def _knn_pallas(xp, bp, n_pad):
    d_feat = xp.shape[1]
    rows = 256
    x2p = jnp.sum(xp * xp, axis=1)
    grid = n_pad // rows
    idx, ok = pl.pallas_call(
        functools.partial(_knn_kernel, n_pad=n_pad, rows=rows),
        grid=(grid,),
        in_specs=[
            pl.BlockSpec((rows, d_feat), lambda i: (i, 0)),
            pl.BlockSpec((d_feat, n_pad), lambda i: (0, 0)),
            pl.BlockSpec((1, n_pad), lambda i: (0, 0)),
            pl.BlockSpec((1, n_pad), lambda i: (0, 0)),
        ],
        out_specs=[
            pl.BlockSpec((rows, _K), lambda i: (i, 0)),
            pl.BlockSpec((rows, _K), lambda i: (i, 0)),
        ],
        out_shape=[
            jax.ShapeDtypeStruct((n_pad, _K), jnp.int32),
            jax.ShapeDtypeStruct((n_pad, _K), jnp.int32),
        ],
    )(xp, xp.T, x2p[None, :], bp[None, :])
    return idx, ok


_CHUNK = 128
_NC = 2
_NS = 16
_NW = _NC * _NS


def _sc_gather_rows(table, idx):
    nw, nc = _NW, _NC
    b = idx.shape[0]
    d = table.shape[1]
    per_w = b // nw
    nchunk = per_w // _CHUNK
    assert nchunk * _CHUNK * nw == b and nchunk % 2 == 0
    idx3 = idx.reshape(nw, nchunk, _CHUNK)
    mesh = plsc.VectorSubcoreMesh(core_axis_name="c", subcore_axis_name="s")

    @functools.partial(
        pl.kernel, mesh=mesh,
        compiler_params=pltpu.CompilerParams(use_tc_tiling_on_sc=False),
        out_type=jax.ShapeDtypeStruct((b, d), table.dtype),
        scratch_types=[
            pltpu.VMEM((nchunk, _CHUNK), jnp.int32),
            pltpu.VMEM((2, _CHUNK, d), table.dtype),
            pltpu.SemaphoreType.DMA,
            pltpu.SemaphoreType.DMA,
            pltpu.SemaphoreType.DMA,
        ],
    )
    def k(table_hbm, idx_hbm, out_hbm, idx_v, rows_v, gsem, wsem0, wsem1):
        wid = lax.axis_index("s") * nc + lax.axis_index("c")
        base = wid * per_w
        pltpu.sync_copy(idx_hbm.at[wid], idx_v)
        wsems = (wsem0, wsem1)

        def gat(j, slot):
            return pltpu.make_async_copy(
                table_hbm.at[idx_v.at[j]], rows_v.at[slot], gsem)

        def wr(j, slot):
            return pltpu.make_async_copy(
                rows_v.at[slot],
                out_hbm.at[pl.ds(base + j * _CHUNK, _CHUNK)], wsems[slot])

        gat(0, 0).start()
        npair = nchunk // 2

        def body(i2, carry):
            j0 = i2 * 2
            gat(j0, 0).wait()
            wr(j0, 0).start()

            @pl.when(j0 >= 1)
            def _():
                wr(j0 - 1, 1).wait()

            gat(j0 + 1, 1).start()
            gat(j0 + 1, 1).wait()
            wr(j0 + 1, 1).start()

            @pl.when(j0 + 2 < nchunk)
            def _():
                wr(j0, 0).wait()
                gat(j0 + 2, 0).start()
            return carry

        lax.fori_loop(0, npair, body, 0)
        wr(nchunk - 2, 0).wait()
        wr(nchunk - 1, 1).wait()

    return k(table, idx3)


def _sc_scatter_add_rows(vals, idx, n_rows):
    nc, ns, nw = _NC, _NS, _NW
    b, d = vals.shape
    per_w = b // nw
    nchunk = per_w // _CHUNK
    idx3 = idx.reshape(nw, nchunk, _CHUNK)
    rows_per_tile = n_rows // ns
    mesh = plsc.VectorSubcoreMesh(core_axis_name="c", subcore_axis_name="s")

    @functools.partial(
        pl.kernel, mesh=mesh,
        compiler_params=pltpu.CompilerParams(use_tc_tiling_on_sc=False),
        out_type=jax.ShapeDtypeStruct((nc, n_rows, d), jnp.float32),
        scratch_types=[
            pltpu.VMEM((nchunk, _CHUNK), jnp.int32),
            pltpu.VMEM((2, _CHUNK, d), jnp.float32),
            pltpu.VMEM_SHARED((n_rows, d), jnp.float32),
            pltpu.SemaphoreType.DMA,
        ],
    )
    def k(vals_hbm, idx_hbm, zeros_hbm, out_hbm, idx_v, rows_v, acc_sh, sem):
        cid = lax.axis_index("c")
        sid = lax.axis_index("s")
        wid = sid * nc + cid
        base = wid * per_w
        pltpu.sync_copy(zeros_hbm.at[pl.ds(sid * rows_per_tile, rows_per_tile)],
                        acc_sh.at[pl.ds(sid * rows_per_tile, rows_per_tile)])
        pltpu.sync_copy(idx_hbm.at[wid], idx_v)
        plsc.subcore_barrier()

        def rd(j, slot):
            return pltpu.make_async_copy(
                vals_hbm.at[pl.ds(base + j * _CHUNK, _CHUNK)],
                rows_v.at[slot], sem)

        rd(0, 0).start()

        def body(i2, carry):
            j0 = i2 * 2
            rd(j0, 0).wait()
            rd(j0 + 1, 1).start()
            pltpu.sync_copy(rows_v.at[0], acc_sh.at[idx_v.at[j0]], add=True)
            rd(j0 + 1, 1).wait()

            @pl.when(j0 + 2 < nchunk)
            def _():
                rd(j0 + 2, 0).start()

            pltpu.sync_copy(rows_v.at[1], acc_sh.at[idx_v.at[j0 + 1]], add=True)
            return carry

        lax.fori_loop(0, nchunk // 2, body, 0)
        plsc.subcore_barrier()
        pltpu.sync_copy(
            acc_sh.at[pl.ds(sid * rows_per_tile, rows_per_tile)],
            out_hbm.at[cid].at[pl.ds(sid * rows_per_tile, rows_per_tile)])

    zeros = jnp.zeros((n_rows, d), jnp.float32)
    out = k(vals, idx3, zeros)
    return out[0] + out[1]


def _edge_mlp_kernel(x_ref, xg_ref, ok_ref, nm_ref,
                     w1_ref, b1_ref, w2_ref, b2_ref,
                     outa_ref, hb_ref, *, vb):
    xv = x_ref[...]
    W1 = w1_ref[...]
    B1 = b1_ref[0]
    W2 = w2_ref[...]
    B2 = b2_ref[0]

    def elu(v):
        vn = jnp.minimum(v, 0.0)
        u = jnp.exp(vn)
        um1 = u - 1.0
        lg = jnp.log(jnp.where(u == 1.0, jnp.e, u))
        em1 = jnp.where(u == 1.0, vn, um1 * (vn / jnp.where(lg == 0.0, 1.0, lg)))
        return jnp.where(v > 0, v, em1)

    def mlp2(f):
        h = jnp.dot(f, W1, preferred_element_type=jnp.float32) + B1
        h = elu(h)
        h = jnp.dot(h, W2, preferred_element_type=jnp.float32) + B2
        return elu(h)

    xg3 = xg_ref[...]
    xv3 = jnp.broadcast_to(xv[None], (_K, vb, _H))
    dA3 = xg3 - xv3
    fA = jnp.concatenate([xv3, dA3], axis=2).reshape(_K * vb, 2 * _H)
    fB = jnp.concatenate([xg3, -dA3], axis=2).reshape(_K * vb, 2 * _H)
    h = mlp2(jnp.concatenate([fA, fB], axis=0))
    hA = h[:_K * vb].reshape(_K, vb, _H)
    hB = h[_K * vb:].reshape(_K, vb, _H)
    okm = ok_ref[...]
    nmm = nm_ref[...]
    outa_ref[...] = jnp.sum(hA * okm[:, :, None], axis=0)
    hb_ref[...] = hB * nmm[:, :, None]


def _edge_mlp(xf, xg, okf, nmf, w1, b1, w2, b2, n_pad, vb=256):
    grid = n_pad // vb
    mw = w1.shape[1]
    outa, hb = pl.pallas_call(
        functools.partial(_edge_mlp_kernel, vb=vb),
        grid=(grid,),
        in_specs=[
            pl.BlockSpec((vb, _H), lambda i: (i, 0)),
            pl.BlockSpec((_K, vb, _H), lambda i: (0, i, 0)),
            pl.BlockSpec((_K, vb), lambda i: (0, i)),
            pl.BlockSpec((_K, vb), lambda i: (0, i)),
            pl.BlockSpec((2 * _H, mw), lambda i: (0, 0)),
            pl.BlockSpec((1, mw), lambda i: (0, 0)),
            pl.BlockSpec((mw, _H), lambda i: (0, 0)),
            pl.BlockSpec((1, _H), lambda i: (0, 0)),
        ],
        out_specs=[
            pl.BlockSpec((vb, _H), lambda i: (i, 0)),
            pl.BlockSpec((_K, vb, _H), lambda i: (0, i, 0)),
        ],
        out_shape=[
            jax.ShapeDtypeStruct((n_pad, _H), jnp.float32),
            jax.ShapeDtypeStruct((_K, n_pad, _H), jnp.float32),
        ],
    )(xf, xg, okf, nmf, w1, b1[None, :], w2, b2[None, :])
    return outa, hb


def _graph_round(xf, bt, w1, b1, w2, b2, n_pad):
    U, OK = _knn_pallas(xf, bt, n_pad)
    UT = U.T
    OKT = OK.T > 0
    uflat = UT.reshape(-1)
    T = jnp.where(OK > 0, U, -1)
    xtab = jnp.concatenate(
        [xf, lax.bitcast_convert_type(T + jnp.int32(0x40000000),
                                      jnp.float32)], axis=1)
    xrows = _sc_gather_rows(xtab, uflat)
    Tg = lax.bitcast_convert_type(xrows[:, _H:],
                                  jnp.int32) - jnp.int32(0x40000000)
    vids = jnp.tile(jnp.arange(n_pad, dtype=jnp.int32), _K)
    MUT = jnp.any(Tg == vids[:, None], axis=1).reshape(_K, n_pad)
    NM = OKT & ~MUT
    okf = OKT.astype(jnp.float32)
    nmf = NM.astype(jnp.float32)

    xg = xrows[:, :_H].reshape(_K, n_pad, _H)
    outa, hb = _edge_mlp(xf, xg, okf, nmf, w1, b1, w2, b2, n_pad)
    accb = _sc_scatter_add_rows(hb.reshape(_K * n_pad, _H), uflat, n_pad)
    hout = outa + accb

    deg = jnp.sum(okf, axis=0) + jax.ops.segment_sum(
        nmf.reshape(-1), uflat, num_segments=n_pad)
    invd = 1.0 / jnp.maximum(deg, 1.0)
    htab = jnp.concatenate(
        [hout, jnp.broadcast_to(invd[:, None], (n_pad, 16))], axis=1)
    hgt = _sc_gather_rows(htab, uflat).reshape(_K, n_pad, _H + 16)
    hg = hgt[:, :, :_H]
    invd_g = hgt[:, :, _H]
    ea = jnp.linalg.norm(hg - hout[None], axis=-1)
    w = ea * (invd[None, :] + invd_g)

    wA = jnp.where(OKT, w, _NEG)
    bestA = jnp.max(wA, axis=0)
    wB = jnp.where(NM, w, _NEG).reshape(-1)
    bestB = jax.ops.segment_max(wB, uflat, num_segments=n_pad)
    best = jnp.maximum(bestA, bestB)
    candA = jnp.max(jnp.where(OKT & (w == best[None, :]), UT, -1), axis=0)
    btab = jnp.broadcast_to(best[:, None], (n_pad, 16))
    bg = _sc_gather_rows(btab, uflat)[:, 0]
    candB = jax.ops.segment_max(
        jnp.where(NM.reshape(-1) & (w.reshape(-1) == bg), vids, -1),
        uflat, num_segments=n_pad)
    prop = jnp.maximum(candA, candB)
    prop = jnp.where(prop < 0, -1, prop)
    idxs = jnp.arange(n_pad, dtype=jnp.int32)
    pp = jnp.where(prop >= 0, prop, idxs)
    mutual = (prop >= 0) & (jnp.take(prop, pp) == idxs)
    return hout, prop, mutual


def _max_pool_pair(prop, mutual, xf, bt, act, n_pad):
    idxs = jnp.arange(n_pad, dtype=jnp.int32)
    am = act & mutual
    partner = jnp.where(am, prop, idxs)
    loser = am & (partner < idxs)
    ptab = jnp.concatenate([
        xf,
        lax.bitcast_convert_type(bt + jnp.int32(0x40000000),
                                 jnp.float32)[:, None],
        act.astype(jnp.float32)[:, None],
        jnp.zeros((n_pad, 14), jnp.float32),
    ], axis=1)
    npad_idx = ((n_pad + 8191) // 8192) * 8192
    pidx = jnp.zeros((npad_idx,), jnp.int32).at[:n_pad].set(partner)
    rows = _sc_gather_rows(ptab, pidx)[:n_pad]
    xpart = rows[:, :_H]
    bpart = lax.bitcast_convert_type(rows[:, _H],
                                     jnp.int32) - jnp.int32(0x40000000)
    apart = rows[:, _H + 1] > 0.5
    xp = jnp.where(loser[:, None], _NEG, jnp.maximum(xf, xpart))
    bp = jnp.where(loser, _IMIN, jnp.maximum(bt, bpart))
    actp = jnp.where(loser, False, act | apart)
    return xp, bp, actp


def kernel(x, batch, datanorm,
           in_w1, in_b1, in_w2, in_b2, in_w3, in_b3,
           c1_w1, c1_b1, c1_w2, c1_b2,
           c2_w1, c2_b1, c2_w2, c2_b2,
           o_w1, o_b1, o_w2, o_b2, o_w3, o_b3):
    n = x.shape[0]
    n_pad = ((n + 1023) // 1024) * 1024
    npd = n_pad - n

    h = datanorm * x
    for W, b in [(in_w1, in_b1), (in_w2, in_b2), (in_w3, in_b3)]:
        h = jax.nn.elu(h @ W + b)

    hp = jnp.concatenate([h, jnp.zeros((npd, _H), jnp.float32)], axis=0)
    bp0 = jnp.concatenate(
        [batch.astype(jnp.int32), -1 - jnp.arange(npd, dtype=jnp.int32)])
    act = jnp.ones((n_pad,), bool)

    h1, prop1, mut1 = _graph_round(hp, bp0, c1_w1, c1_b1, c1_w2, c1_b2, n_pad)
    h1p, b1p, act1 = _max_pool_pair(prop1, mut1, h1, bp0, act, n_pad)

    hk = jnp.where(act1[:, None], h1p, 0.0)
    bk = jnp.where(act1, b1p,
                   _NUM_GRAPHS + jnp.arange(n_pad, dtype=jnp.int32))
    bk = jnp.where(jnp.arange(n_pad) < n, bk,
                   -1 - jnp.arange(n_pad, dtype=jnp.int32))

    h2, prop2, mut2 = _graph_round(hk, bk, c2_w1, c2_b1, c2_w2, c2_b2, n_pad)
    h2p, b2p, act2 = _max_pool_pair(prop2, mut2, h2, bk, act1, n_pad)

    hs, bs, acts = h2p[:n], b2p[:n], act2[:n]
    hf = jnp.where(acts[:, None], hs, _NEG)
    bf = jnp.where(acts, bs, 0)
    g = jax.ops.segment_max(hf, bf, num_segments=_NUM_GRAPHS)
    g = jnp.where(jnp.isfinite(g), g, 0.0)
    z = jax.nn.elu(g @ o_w1 + o_b1)
    z = jax.nn.elu(z @ o_w2 + o_b2)
    return z @ o_w3 + o_b3


# speedup vs baseline: 1.0829x; 1.0587x over previous
import functools

import jax
import jax.numpy as jnp
import numpy as np
from jax import lax
from jax.experimental import pallas as pl
from jax.experimental.pallas import tpu as pltpu
from jax.experimental.pallas import tpu_sc as plsc

_K = 16
_NUM_GRAPHS = 16
_H = 64
_NEG = float('-inf')
_IMIN = int(np.iinfo(np.int32).min)


def _knn_kernel(x_ref, xt_ref, x2_ref, batch_ref, idx_ref, ok_ref, *, n_pad, rows):
    i = pl.program_id(0)
    xb = x_ref[...]
    mm = jax.lax.dot_general(
        xb, xt_ref[...], (((1,), (0,)), ((), ())),
        preferred_element_type=jnp.float32)
    x2_row = x2_ref[...]
    x2b = x2_ref[0, pl.ds(i * rows, rows)]
    d = (x2b[:, None] - 2.0 * mm) + x2_row
    cols = jax.lax.broadcasted_iota(jnp.int32, (rows, n_pad), 1)
    row_ids = i * rows + jax.lax.broadcasted_iota(jnp.int32, (rows, n_pad), 0)
    bb = batch_ref[0, pl.ds(i * rows, rows)]
    valid = (bb[:, None] == batch_ref[...]) & (row_ids != cols)
    d = jnp.where(valid, d, jnp.inf)
    nv = jnp.sum(valid.astype(jnp.int32), axis=1)
    for t in range(_K):
        a = jnp.argmin(d, axis=1).astype(jnp.int32)
        idx_ref[:, t] = a
        ok_ref[:, t] = (t < nv).astype(jnp.int32)
        if t + 1 < _K:
            d = jnp.where(cols == a[:, None], jnp.inf, d)


def _knn_pallas(xp, bp, n_pad):
    d_feat = xp.shape[1]
    rows = 512
    x2p = jnp.sum(xp * xp, axis=1)
    grid = n_pad // rows
    idx, ok = pl.pallas_call(
        functools.partial(_knn_kernel, n_pad=n_pad, rows=rows),
        grid=(grid,),
        in_specs=[
            pl.BlockSpec((rows, d_feat), lambda i: (i, 0)),
            pl.BlockSpec((d_feat, n_pad), lambda i: (0, 0)),
            pl.BlockSpec((1, n_pad), lambda i: (0, 0)),
            pl.BlockSpec((1, n_pad), lambda i: (0, 0)),
        ],
        out_specs=[
            pl.BlockSpec((rows, _K), lambda i: (i, 0)),
            pl.BlockSpec((rows, _K), lambda i: (i, 0)),
        ],
        out_shape=[
            jax.ShapeDtypeStruct((n_pad, _K), jnp.int32),
            jax.ShapeDtypeStruct((n_pad, _K), jnp.int32),
        ],
    )(xp, xp.T, x2p[None, :], bp[None, :])
    return idx, ok


_CHUNK = 128
_NC = 2
_NS = 16
_NW = _NC * _NS


def _sc_gather_rows(table, idx):
    nw, nc = _NW, _NC
    b = idx.shape[0]
    d = table.shape[1]
    per_w = b // nw
    nchunk = per_w // _CHUNK
    assert nchunk * _CHUNK * nw == b and nchunk % 2 == 0
    idx3 = idx.reshape(nw, nchunk, _CHUNK)
    mesh = plsc.VectorSubcoreMesh(core_axis_name="c", subcore_axis_name="s")

    @functools.partial(
        pl.kernel, mesh=mesh,
        compiler_params=pltpu.CompilerParams(use_tc_tiling_on_sc=False),
        out_type=jax.ShapeDtypeStruct((b, d), table.dtype),
        scratch_types=[
            pltpu.VMEM((nchunk, _CHUNK), jnp.int32),
            pltpu.VMEM((2, _CHUNK, d), table.dtype),
            pltpu.SemaphoreType.DMA,
            pltpu.SemaphoreType.DMA,
            pltpu.SemaphoreType.DMA,
        ],
    )
    def k(table_hbm, idx_hbm, out_hbm, idx_v, rows_v, gsem, wsem0, wsem1):
        wid = lax.axis_index("s") * nc + lax.axis_index("c")
        base = wid * per_w
        pltpu.sync_copy(idx_hbm.at[wid], idx_v)
        wsems = (wsem0, wsem1)

        def gat(j, slot):
            return pltpu.make_async_copy(
                table_hbm.at[idx_v.at[j]], rows_v.at[slot], gsem)

        def wr(j, slot):
            return pltpu.make_async_copy(
                rows_v.at[slot],
                out_hbm.at[pl.ds(base + j * _CHUNK, _CHUNK)], wsems[slot])

        gat(0, 0).start()
        npair = nchunk // 2

        def body(i2, carry):
            j0 = i2 * 2
            gat(j0, 0).wait()
            wr(j0, 0).start()

            @pl.when(j0 >= 1)
            def _():
                wr(j0 - 1, 1).wait()

            gat(j0 + 1, 1).start()
            gat(j0 + 1, 1).wait()
            wr(j0 + 1, 1).start()

            @pl.when(j0 + 2 < nchunk)
            def _():
                wr(j0, 0).wait()
                gat(j0 + 2, 0).start()
            return carry

        lax.fori_loop(0, npair, body, 0)
        wr(nchunk - 2, 0).wait()
        wr(nchunk - 1, 1).wait()

    return k(table, idx3)


def _sc_scatter_add_rows(vals, idx, n_rows):
    nc, ns, nw = _NC, _NS, _NW
    b, d = vals.shape
    per_w = b // nw
    nchunk = per_w // _CHUNK
    idx3 = idx.reshape(nw, nchunk, _CHUNK)
    rows_per_tile = n_rows // ns
    mesh = plsc.VectorSubcoreMesh(core_axis_name="c", subcore_axis_name="s")

    @functools.partial(
        pl.kernel, mesh=mesh,
        compiler_params=pltpu.CompilerParams(use_tc_tiling_on_sc=False),
        out_type=jax.ShapeDtypeStruct((nc, n_rows, d), jnp.float32),
        scratch_types=[
            pltpu.VMEM((nchunk, _CHUNK), jnp.int32),
            pltpu.VMEM((2, _CHUNK, d), jnp.float32),
            pltpu.VMEM_SHARED((n_rows, d), jnp.float32),
            pltpu.SemaphoreType.DMA,
        ],
    )
    def k(vals_hbm, idx_hbm, zeros_hbm, out_hbm, idx_v, rows_v, acc_sh, sem):
        cid = lax.axis_index("c")
        sid = lax.axis_index("s")
        wid = sid * nc + cid
        base = wid * per_w
        pltpu.sync_copy(zeros_hbm.at[pl.ds(sid * rows_per_tile, rows_per_tile)],
                        acc_sh.at[pl.ds(sid * rows_per_tile, rows_per_tile)])
        pltpu.sync_copy(idx_hbm.at[wid], idx_v)
        plsc.subcore_barrier()

        def rd(j, slot):
            return pltpu.make_async_copy(
                vals_hbm.at[pl.ds(base + j * _CHUNK, _CHUNK)],
                rows_v.at[slot], sem)

        rd(0, 0).start()

        def body(i2, carry):
            j0 = i2 * 2
            rd(j0, 0).wait()
            rd(j0 + 1, 1).start()
            pltpu.sync_copy(rows_v.at[0], acc_sh.at[idx_v.at[j0]], add=True)
            rd(j0 + 1, 1).wait()

            @pl.when(j0 + 2 < nchunk)
            def _():
                rd(j0 + 2, 0).start()

            pltpu.sync_copy(rows_v.at[1], acc_sh.at[idx_v.at[j0 + 1]], add=True)
            return carry

        lax.fori_loop(0, nchunk // 2, body, 0)
        plsc.subcore_barrier()
        pltpu.sync_copy(
            acc_sh.at[pl.ds(sid * rows_per_tile, rows_per_tile)],
            out_hbm.at[cid].at[pl.ds(sid * rows_per_tile, rows_per_tile)])

    zeros = jnp.zeros((n_rows, d), jnp.float32)
    out = k(vals, idx3, zeros)
    return out[0] + out[1]


def _edge_mlp_kernel(x_ref, xg_ref, ok_ref, nm_ref,
                     w1_ref, b1_ref, w2_ref, b2_ref,
                     outa_ref, hb_ref, *, vb):
    xv = x_ref[...]
    W1 = w1_ref[...]
    B1 = b1_ref[0]
    W2 = w2_ref[...]
    B2 = b2_ref[0]

    def elu(v):
        vn = jnp.minimum(v, 0.0)
        u = jnp.exp(vn)
        um1 = u - 1.0
        lg = jnp.log(jnp.where(u == 1.0, jnp.e, u))
        em1 = jnp.where(u == 1.0, vn, um1 * (vn / jnp.where(lg == 0.0, 1.0, lg)))
        return jnp.where(v > 0, v, em1)

    def mlp2(f):
        h = jnp.dot(f, W1, preferred_element_type=jnp.float32) + B1
        h = elu(h)
        h = jnp.dot(h, W2, preferred_element_type=jnp.float32) + B2
        return elu(h)

    xg3 = xg_ref[...]
    xv3 = jnp.broadcast_to(xv[None], (_K, vb, _H))
    dA3 = xg3 - xv3
    fA = jnp.concatenate([xv3, dA3], axis=2).reshape(_K * vb, 2 * _H)
    fB = jnp.concatenate([xg3, -dA3], axis=2).reshape(_K * vb, 2 * _H)
    h = mlp2(jnp.concatenate([fA, fB], axis=0))
    hA = h[:_K * vb].reshape(_K, vb, _H)
    hB = h[_K * vb:].reshape(_K, vb, _H)
    okm = ok_ref[...]
    nmm = nm_ref[...]
    outa_ref[...] = jnp.sum(hA * okm[:, :, None], axis=0)
    hb_ref[...] = hB * nmm[:, :, None]


def _edge_mlp(xf, xg, okf, nmf, w1, b1, w2, b2, n_pad, vb=256):
    grid = n_pad // vb
    mw = w1.shape[1]
    outa, hb = pl.pallas_call(
        functools.partial(_edge_mlp_kernel, vb=vb),
        grid=(grid,),
        in_specs=[
            pl.BlockSpec((vb, _H), lambda i: (i, 0)),
            pl.BlockSpec((_K, vb, _H), lambda i: (0, i, 0)),
            pl.BlockSpec((_K, vb), lambda i: (0, i)),
            pl.BlockSpec((_K, vb), lambda i: (0, i)),
            pl.BlockSpec((2 * _H, mw), lambda i: (0, 0)),
            pl.BlockSpec((1, mw), lambda i: (0, 0)),
            pl.BlockSpec((mw, _H), lambda i: (0, 0)),
            pl.BlockSpec((1, _H), lambda i: (0, 0)),
        ],
        out_specs=[
            pl.BlockSpec((vb, _H), lambda i: (i, 0)),
            pl.BlockSpec((_K, vb, _H), lambda i: (0, i, 0)),
        ],
        out_shape=[
            jax.ShapeDtypeStruct((n_pad, _H), jnp.float32),
            jax.ShapeDtypeStruct((_K, n_pad, _H), jnp.float32),
        ],
    )(xf, xg, okf, nmf, w1, b1[None, :], w2, b2[None, :])
    return outa, hb


def _graph_round(xf, bt, w1, b1, w2, b2, n_pad):
    U, OK = _knn_pallas(xf, bt, n_pad)
    UT = U.T
    OKT = OK.T > 0
    uflat = UT.reshape(-1)
    T = jnp.where(OK > 0, U, -1)
    Tg = _sc_gather_rows(T, uflat)
    vids = jnp.tile(jnp.arange(n_pad, dtype=jnp.int32), _K)
    MUT = jnp.any(Tg == vids[:, None], axis=1).reshape(_K, n_pad)
    NM = OKT & ~MUT
    okf = OKT.astype(jnp.float32)
    nmf = NM.astype(jnp.float32)

    xg = _sc_gather_rows(xf, uflat).reshape(_K, n_pad, _H)
    outa, hb = _edge_mlp(xf, xg, okf, nmf, w1, b1, w2, b2, n_pad)
    accb = _sc_scatter_add_rows(hb.reshape(_K * n_pad, _H), uflat, n_pad)
    hout = outa + accb

    deg = jnp.sum(okf, axis=0) + jax.ops.segment_sum(
        nmf.reshape(-1), uflat, num_segments=n_pad)
    invd = 1.0 / jnp.maximum(deg, 1.0)
    htab = jnp.concatenate(
        [hout, jnp.broadcast_to(invd[:, None], (n_pad, 16))], axis=1)
    hgt = _sc_gather_rows(htab, uflat).reshape(_K, n_pad, _H + 16)
    hg = hgt[:, :, :_H]
    invd_g = hgt[:, :, _H]
    ea = jnp.linalg.norm(hg - hout[None], axis=-1)
    w = ea * (invd[None, :] + invd_g)

    wA = jnp.where(OKT, w, _NEG)
    bestA = jnp.max(wA, axis=0)
    wB = jnp.where(NM, w, _NEG).reshape(-1)
    bestB = jax.ops.segment_max(wB, uflat, num_segments=n_pad)
    best = jnp.maximum(bestA, bestB)
    candA = jnp.max(jnp.where(OKT & (w == best[None, :]), UT, -1), axis=0)
    btab = jnp.broadcast_to(best[:, None], (n_pad, 16))
    bg = _sc_gather_rows(btab, uflat)[:, 0]
    candB = jax.ops.segment_max(
        jnp.where(NM.reshape(-1) & (w.reshape(-1) == bg), vids, -1),
        uflat, num_segments=n_pad)
    prop = jnp.maximum(candA, candB)
    prop = jnp.where(prop < 0, -1, prop)
    idxs = jnp.arange(n_pad, dtype=jnp.int32)
    pp = jnp.where(prop >= 0, prop, idxs)
    mutual = (prop >= 0) & (jnp.take(prop, pp) == idxs)
    return hout, prop, mutual


def _max_pool_pair(prop, mutual, xf, bt, act, n_pad):
    idxs = jnp.arange(n_pad, dtype=jnp.int32)
    am = act & mutual
    partner = jnp.where(am, prop, idxs)
    loser = am & (partner < idxs)
    ptab = jnp.concatenate([
        xf,
        lax.bitcast_convert_type(bt, jnp.float32)[:, None],
        act.astype(jnp.float32)[:, None],
        jnp.zeros((n_pad, 14), jnp.float32),
    ], axis=1)
    npad_idx = ((n_pad + 8191) // 8192) * 8192
    pidx = jnp.zeros((npad_idx,), jnp.int32).at[:n_pad].set(partner)
    rows = _sc_gather_rows(ptab, pidx)[:n_pad]
    xpart = rows[:, :_H]
    bpart = lax.bitcast_convert_type(rows[:, _H], jnp.int32)
    apart = rows[:, _H + 1] > 0.5
    xp = jnp.where(loser[:, None], _NEG, jnp.maximum(xf, xpart))
    bp = jnp.where(loser, _IMIN, jnp.maximum(bt, bpart))
    actp = jnp.where(loser, False, act | apart)
    return xp, bp, actp


def kernel(x, batch, datanorm,
           in_w1, in_b1, in_w2, in_b2, in_w3, in_b3,
           c1_w1, c1_b1, c1_w2, c1_b2,
           c2_w1, c2_b1, c2_w2, c2_b2,
           o_w1, o_b1, o_w2, o_b2, o_w3, o_b3):
    n = x.shape[0]
    n_pad = ((n + 1023) // 1024) * 1024
    npd = n_pad - n

    h = datanorm * x
    for W, b in [(in_w1, in_b1), (in_w2, in_b2), (in_w3, in_b3)]:
        h = jax.nn.elu(h @ W + b)

    hp = jnp.concatenate([h, jnp.zeros((npd, _H), jnp.float32)], axis=0)
    bp0 = jnp.concatenate(
        [batch.astype(jnp.int32), -1 - jnp.arange(npd, dtype=jnp.int32)])
    act = jnp.ones((n_pad,), bool)

    h1, prop1, mut1 = _graph_round(hp, bp0, c1_w1, c1_b1, c1_w2, c1_b2, n_pad)
    h1p, b1p, act1 = _max_pool_pair(prop1, mut1, h1, bp0, act, n_pad)

    hk = jnp.where(act1[:, None], h1p, 0.0)
    bk = jnp.where(act1, b1p,
                   _NUM_GRAPHS + jnp.arange(n_pad, dtype=jnp.int32))
    bk = jnp.where(jnp.arange(n_pad) < n, bk,
                   -1 - jnp.arange(n_pad, dtype=jnp.int32))

    h2, prop2, mut2 = _graph_round(hk, bk, c2_w1, c2_b1, c2_w2, c2_b2, n_pad)
    h2p, b2p, act2 = _max_pool_pair(prop2, mut2, h2, bk, act1, n_pad)

    hs, bs, acts = h2p[:n], b2p[:n], act2[:n]
    hf = jnp.where(acts[:, None], hs, _NEG)
    bf = jnp.where(acts, bs, 0)
    g = jax.ops.segment_max(hf, bf, num_segments=_NUM_GRAPHS)
    g = jnp.where(jnp.isfinite(g), g, 0.0)
    z = jax.nn.elu(g @ o_w1 + o_b1)
    z = jax.nn.elu(z @ o_w2 + o_b2)
    return z @ o_w3 + o_b3
